# Initial kernel scaffold; baseline (speedup 1.0000x reference)
import jax
import jax.numpy as jnp
from jax.experimental import pallas as pl


def kernel(x, table):
    raise NotImplementedError("write your pallas kernel here")


# speedup vs baseline: 1.4581x; 1.4581x over previous
import functools

import jax
import jax.numpy as jnp
from jax import lax
from jax.experimental import pallas as pl
from jax.experimental.pallas import tpu as pltpu
from jax.experimental.pallas import tpu_sc as plsc

NC, NS = 2, 16
NW = NC * NS
ROW = 128
G = 8


@functools.partial(jax.jit, static_argnames=("rows_per_w",))
def _emb_lookup(xf, table, rows_per_w):
    n_rows = xf.shape[0]
    d = table.shape[1]
    n = n_rows * ROW
    n_chunks = rows_per_w // G

    mesh = plsc.VectorSubcoreMesh(
        core_axis_name="c", subcore_axis_name="s", num_cores=NC, num_subcores=NS
    )

    @functools.partial(
        pl.kernel,
        mesh=mesh,
        compiler_params=pltpu.CompilerParams(use_tc_tiling_on_sc=False),
        out_type=jax.ShapeDtypeStruct((n, d), jnp.float32),
        scratch_types=[
            pltpu.VMEM((G, ROW), jnp.int32),
            pltpu.VMEM((G * ROW, d), jnp.float32),
            pltpu.SemaphoreType.DMA,
        ],
    )
    def body(x_hbm, table_hbm, out_hbm, idx_v, rows_v, sem):
        wid = lax.axis_index("s") * NC + lax.axis_index("c")
        base_row = wid * rows_per_w

        def chunk(g, carry):
            r0 = base_row + g * G
            pltpu.sync_copy(x_hbm.at[pl.ds(r0, G)], idx_v)
            copies = [
                pltpu.async_copy(
                    table_hbm.at[idx_v.at[j]],
                    rows_v.at[pl.ds(j * ROW, ROW)],
                    sem,
                )
                for j in range(G)
            ]
            for c in copies:
                c.wait()
            pltpu.sync_copy(rows_v, out_hbm.at[pl.ds(r0 * ROW, G * ROW)])
            return carry

        lax.fori_loop(0, n_chunks, chunk, 0)

    return body(xf, table)


def kernel(x, table):
    b, h = x.shape
    d = table.shape[1]
    n = b * h
    xf = x.reshape(n // ROW, ROW)
    rows_per_w = (n // ROW) // NW
    out = _emb_lookup(xf, table, rows_per_w)
    return out.reshape(b, h, d)


# speedup vs baseline: 1.4934x; 1.0242x over previous
import functools

import jax
import jax.numpy as jnp
from jax import lax
from jax.experimental import pallas as pl
from jax.experimental.pallas import tpu as pltpu
from jax.experimental.pallas import tpu_sc as plsc

NC, NS = 2, 16
NW = NC * NS
ROW = 128
G = 10


@functools.partial(jax.jit, static_argnames=("rows_per_w",))
def _emb_lookup(xf, table, rows_per_w):
    n_rows = xf.shape[0]
    d = table.shape[1]
    n = n_rows * ROW
    n_chunks = rows_per_w // G
    n_outer = n_chunks // 2

    mesh = plsc.VectorSubcoreMesh(
        core_axis_name="c", subcore_axis_name="s", num_cores=NC, num_subcores=NS
    )

    @functools.partial(
        pl.kernel,
        mesh=mesh,
        compiler_params=pltpu.CompilerParams(use_tc_tiling_on_sc=False),
        out_type=jax.ShapeDtypeStruct((n, d), jnp.float32),
        scratch_types=[
            pltpu.VMEM((2, G, ROW), jnp.int32),
            pltpu.VMEM((2, G * ROW, d), jnp.float32),
            pltpu.SemaphoreType.DMA,
            pltpu.SemaphoreType.DMA,
            pltpu.SemaphoreType.DMA,
            pltpu.SemaphoreType.DMA,
        ],
    )
    def body(x_hbm, table_hbm, out_hbm, idx_v, rows_v, sg0, sg1, so0, so1):
        wid = lax.axis_index("s") * NC + lax.axis_index("c")
        base_row = wid * rows_per_w
        sg = [sg0, sg1]
        so = [so0, so1]

        def stage_idx(g, b):
            pltpu.sync_copy(x_hbm.at[pl.ds(base_row + g * G, G)], idx_v.at[b])

        def fire_gathers(b):
            for j in range(G):
                pltpu.async_copy(
                    table_hbm.at[idx_v.at[b].at[j]],
                    rows_v.at[b].at[pl.ds(j * ROW, ROW)],
                    sg[b],
                )

        def drain_gathers(b):
            for j in range(G):
                pltpu.make_async_copy(
                    table_hbm.at[idx_v.at[b].at[j]],
                    rows_v.at[b].at[pl.ds(j * ROW, ROW)],
                    sg[b],
                ).wait()

        def start_out(g, b):
            pltpu.async_copy(
                rows_v.at[b],
                out_hbm.at[pl.ds((base_row + g * G) * ROW, G * ROW)],
                so[b],
            )

        def wait_out(b):
            pltpu.make_async_copy(
                rows_v.at[b],
                out_hbm.at[pl.ds(base_row * ROW, G * ROW)],
                so[b],
            ).wait()

        stage_idx(0, 0)
        fire_gathers(0)
        stage_idx(1, 1)

        def outer(t, carry):
            g0 = 2 * t
            drain_gathers(0)
            start_out(g0, 0)
            lax.cond(t >= 1, lambda: wait_out(1), lambda: None)
            fire_gathers(1)
            lax.cond(
                t + 1 < n_outer, lambda: stage_idx(g0 + 2, 0), lambda: None
            )
            drain_gathers(1)
            start_out(g0 + 1, 1)
            wait_out(0)
            lax.cond(t + 1 < n_outer, lambda: fire_gathers(0), lambda: None)
            lax.cond(
                t + 1 < n_outer, lambda: stage_idx(g0 + 3, 1), lambda: None
            )
            return carry

        lax.fori_loop(0, n_outer, outer, 0)
        wait_out(1)

    return body(xf, table)


def kernel(x, table):
    b, h = x.shape
    d = table.shape[1]
    n = b * h
    xf = x.reshape(n // ROW, ROW)
    rows_per_w = (n // ROW) // NW
    out = _emb_lookup(xf, table, rows_per_w)
    return out.reshape(b, h, d)


# speedup vs baseline: 1.4938x; 1.0002x over previous
import functools

import jax
import jax.numpy as jnp
from jax import lax
from jax.experimental import pallas as pl
from jax.experimental.pallas import tpu as pltpu
from jax.experimental.pallas import tpu_sc as plsc

NC, NS = 2, 16
NW = NC * NS
ROW = 128
G = 10


@functools.partial(jax.jit, static_argnames=("rows_per_w",))
def _emb_lookup(xf, table, rows_per_w):
    n = xf.shape[0]
    d = table.shape[1]
    n_chunks = rows_per_w // G
    n_outer = n_chunks // 2

    mesh = plsc.VectorSubcoreMesh(
        core_axis_name="c", subcore_axis_name="s", num_cores=NC, num_subcores=NS
    )

    @functools.partial(
        pl.kernel,
        mesh=mesh,
        compiler_params=pltpu.CompilerParams(use_tc_tiling_on_sc=False),
        out_type=jax.ShapeDtypeStruct((n, d), jnp.float32),
        scratch_types=[
            pltpu.VMEM((2, G * ROW), jnp.int32),
            pltpu.VMEM((2, G * ROW, d), jnp.float32),
            pltpu.SemaphoreType.DMA,
            pltpu.SemaphoreType.DMA,
            pltpu.SemaphoreType.DMA,
            pltpu.SemaphoreType.DMA,
        ],
    )
    def body(x_hbm, table_hbm, out_hbm, idx_v, rows_v, sg0, sg1, so0, so1):
        wid = lax.axis_index("s") * NC + lax.axis_index("c")
        base_row = wid * rows_per_w
        sg = [sg0, sg1]
        so = [so0, so1]

        def stage_idx(g, b):
            pltpu.sync_copy(
                x_hbm.at[pl.ds((base_row + g * G) * ROW, G * ROW)], idx_v.at[b]
            )

        def fire_gathers(b):
            pltpu.async_copy(
                table_hbm.at[idx_v.at[b]],
                rows_v.at[b],
                sg[b],
            )

        def drain_gathers(b):
            pltpu.make_async_copy(
                table_hbm.at[idx_v.at[b]],
                rows_v.at[b],
                sg[b],
            ).wait()

        def start_out(g, b):
            pltpu.async_copy(
                rows_v.at[b],
                out_hbm.at[pl.ds((base_row + g * G) * ROW, G * ROW)],
                so[b],
            )

        def wait_out(b):
            pltpu.make_async_copy(
                rows_v.at[b],
                out_hbm.at[pl.ds(base_row * ROW, G * ROW)],
                so[b],
            ).wait()

        stage_idx(0, 0)
        fire_gathers(0)
        stage_idx(1, 1)

        def outer(t, carry):
            g0 = 2 * t
            drain_gathers(0)
            start_out(g0, 0)
            lax.cond(t >= 1, lambda: wait_out(1), lambda: None)
            fire_gathers(1)
            lax.cond(
                t + 1 < n_outer, lambda: stage_idx(g0 + 2, 0), lambda: None
            )
            drain_gathers(1)
            start_out(g0 + 1, 1)
            wait_out(0)
            lax.cond(t + 1 < n_outer, lambda: fire_gathers(0), lambda: None)
            lax.cond(
                t + 1 < n_outer, lambda: stage_idx(g0 + 3, 1), lambda: None
            )
            return carry

        lax.fori_loop(0, n_outer, outer, 0)
        wait_out(1)

    return body(xf, table)


def kernel(x, table):
    b, h = x.shape
    d = table.shape[1]
    n = b * h
    xf = x.reshape(n)
    rows_per_w = (n // ROW) // NW
    out = _emb_lookup(xf, table, rows_per_w)
    return out.reshape(b, h, d)
